# baseline (speedup 1.0000x reference)
import functools

import jax
import jax.numpy as jnp
import numpy as np
from jax import lax
from jax.experimental import pallas as pl
from jax.experimental.pallas import tpu as pltpu
from jax.experimental.pallas import tpu_sc as plsc

LATENT = 64
BATCH = 1024
TWOB = 2 * BATCH
LMBDA = 0.1
_LOG2 = float(np.log(2.0))

_info = plsc.get_sparse_core_info()
_NC, _NS = _info.num_cores, _info.num_subcores
_NW = _NC * _NS
_BPW = TWOB // _NW


def _gather_body(ent_hbm, rel_hbm, bh_hbm, bt_hbm, br_hbm,
                 eh_out, et_out, er_out,
                 idxh_v, idxt_v, idxr_v, rh_v, rt_v, rr_v,
                 sem_h, sem_t, sem_r):
    wid = lax.axis_index("s") * _NC + lax.axis_index("c")
    base = wid * _BPW
    pltpu.sync_copy(bh_hbm.at[pl.ds(base, _BPW)], idxh_v)
    pltpu.sync_copy(bt_hbm.at[pl.ds(base, _BPW)], idxt_v)
    pltpu.sync_copy(br_hbm.at[pl.ds(base, _BPW)], idxr_v)
    ch = pltpu.async_copy(ent_hbm.at[idxh_v], rh_v, sem_h)
    ct = pltpu.async_copy(ent_hbm.at[idxt_v], rt_v, sem_t)
    cr = pltpu.async_copy(rel_hbm.at[idxr_v], rr_v, sem_r)
    ch.wait()
    ct.wait()
    cr.wait()
    pltpu.sync_copy(rh_v, eh_out.at[pl.ds(base, _BPW)])
    pltpu.sync_copy(rt_v, et_out.at[pl.ds(base, _BPW)])
    pltpu.sync_copy(rr_v, er_out.at[pl.ds(base, _BPW)])


_gather3 = functools.partial(
    pl.kernel,
    out_type=[
        jax.ShapeDtypeStruct((TWOB, LATENT), jnp.float32),
        jax.ShapeDtypeStruct((TWOB, LATENT), jnp.float32),
        jax.ShapeDtypeStruct((TWOB, LATENT), jnp.float32),
    ],
    mesh=plsc.VectorSubcoreMesh(core_axis_name="c", subcore_axis_name="s"),
    compiler_params=pltpu.CompilerParams(use_tc_tiling_on_sc=False),
    scratch_types=[
        pltpu.VMEM((_BPW,), jnp.int32),
        pltpu.VMEM((_BPW,), jnp.int32),
        pltpu.VMEM((_BPW,), jnp.int32),
        pltpu.VMEM((_BPW, LATENT), jnp.float32),
        pltpu.VMEM((_BPW, LATENT), jnp.float32),
        pltpu.VMEM((_BPW, LATENT), jnp.float32),
        pltpu.SemaphoreType.DMA,
        pltpu.SemaphoreType.DMA,
        pltpu.SemaphoreType.DMA,
    ],
)(_gather_body)


def _finish_body(eh_ref, et_ref, er_ref, take2_ref, loss_ref, nsc_ref):
    eh = eh_ref[...]
    et = et_ref[...]
    er = er_ref[...]
    s = jnp.sum(eh * et * er, axis=1)
    nsc_ref[...] = s[BATCH:]
    a = jnp.abs(s)
    sp_pair = a + 2.0 * jnp.log1p(jnp.exp(-a))
    contrib = jnp.where(take2_ref[...] > 0, sp_pair, 2.0 * _LOG2)
    loss_main = jnp.sum(contrib) / (4.0 * BATCH)
    ssq = jnp.sum(eh * eh) + jnp.sum(et * et) + jnp.sum(er * er)
    regul = ssq / float(TWOB * LATENT)
    loss_ref[...] = jnp.broadcast_to(loss_main + LMBDA * regul, (1, 1))


def kernel(ent_embeddings, rel_embeddings, pos_h, pos_r, pos_t,
           neg_h, neg_r, neg_t, take):
    bh = jnp.concatenate([pos_h, neg_h]).astype(jnp.int32)
    bt = jnp.concatenate([pos_t, neg_t]).astype(jnp.int32)
    br = jnp.concatenate([pos_r, neg_r]).astype(jnp.int32)
    take2 = jnp.concatenate([take, take]).astype(jnp.float32)

    eh, et, er = _gather3(ent_embeddings, rel_embeddings, bh, bt, br)

    loss2d, n_score = pl.pallas_call(
        _finish_body,
        out_shape=[
            jax.ShapeDtypeStruct((1, 1), jnp.float32),
            jax.ShapeDtypeStruct((BATCH,), jnp.float32),
        ],
    )(eh, et, er, take2)
    return loss2d[0, 0], n_score


# speedup vs baseline: 1.7235x; 1.7235x over previous
import functools

import jax
import jax.numpy as jnp
import numpy as np
from jax import lax
from jax.experimental import pallas as pl
from jax.experimental.pallas import tpu as pltpu
from jax.experimental.pallas import tpu_sc as plsc

LATENT = 64
BATCH = 1024
TWOB = 2 * BATCH
LMBDA = 0.1
_LOG2 = float(np.log(2.0))

_info = plsc.get_sparse_core_info()
_NC, _NS = _info.num_cores, _info.num_subcores
_NW = _NC * _NS
_BPW = TWOB // _NW


def _gather_body(ent_hbm, rel_hbm, bh_hbm, bt_hbm, br_hbm,
                 eh_out, et_out, er_out,
                 idxh_v, idxt_v, idxr_v,
                 rh_v, rt_v, rr_v, sem):
    wid = lax.axis_index("s") * _NC + lax.axis_index("c")
    base = wid * _BPW
    pltpu.sync_copy(bh_hbm.at[pl.ds(base, _BPW)], idxh_v)
    pltpu.sync_copy(bt_hbm.at[pl.ds(base, _BPW)], idxt_v)
    pltpu.sync_copy(br_hbm.at[pl.ds(base, _BPW)], idxr_v)
    copies = []
    for g in range(_BPW // 16):
        vh = idxh_v[pl.ds(g * 16, 16)]
        vt = idxt_v[pl.ds(g * 16, 16)]
        vr = idxr_v[pl.ds(g * 16, 16)]
        for l in range(16):
            dst = pl.ds(g * 16 + l, 1)
            copies.append(pltpu.async_copy(
                ent_hbm.at[pl.ds(vh[l], 1)], rh_v.at[dst], sem))
            copies.append(pltpu.async_copy(
                ent_hbm.at[pl.ds(vt[l], 1)], rt_v.at[dst], sem))
            copies.append(pltpu.async_copy(
                rel_hbm.at[pl.ds(vr[l], 1)], rr_v.at[dst], sem))
    for c in copies:
        c.wait()
    pltpu.sync_copy(rh_v, eh_out.at[pl.ds(base, _BPW)])
    pltpu.sync_copy(rt_v, et_out.at[pl.ds(base, _BPW)])
    pltpu.sync_copy(rr_v, er_out.at[pl.ds(base, _BPW)])


_gather3 = functools.partial(
    pl.kernel,
    out_type=[
        jax.ShapeDtypeStruct((TWOB, LATENT), jnp.float32),
        jax.ShapeDtypeStruct((TWOB, LATENT), jnp.float32),
        jax.ShapeDtypeStruct((TWOB, LATENT), jnp.float32),
    ],
    mesh=plsc.VectorSubcoreMesh(core_axis_name="c", subcore_axis_name="s"),
    scratch_types=[
        pltpu.VMEM((_BPW,), jnp.int32),
        pltpu.VMEM((_BPW,), jnp.int32),
        pltpu.VMEM((_BPW,), jnp.int32),
        pltpu.VMEM((_BPW, LATENT), jnp.float32),
        pltpu.VMEM((_BPW, LATENT), jnp.float32),
        pltpu.VMEM((_BPW, LATENT), jnp.float32),
        pltpu.SemaphoreType.DMA,
    ],
)(_gather_body)


def _finish_body(eh_ref, et_ref, er_ref, take2_ref, loss_ref, nsc_ref):
    eh = eh_ref[...]
    et = et_ref[...]
    er = er_ref[...]
    s = jnp.sum(eh * et * er, axis=1)
    nsc_ref[...] = s[BATCH:]
    a = jnp.abs(s)
    sp_pair = a + 2.0 * jnp.log1p(jnp.exp(-a))
    contrib = jnp.where(take2_ref[...] > 0, sp_pair, 2.0 * _LOG2)
    loss_main = jnp.sum(contrib) / (4.0 * BATCH)
    ssq = jnp.sum(eh * eh) + jnp.sum(et * et) + jnp.sum(er * er)
    regul = ssq / float(TWOB * LATENT)
    loss_ref[...] = jnp.broadcast_to(loss_main + LMBDA * regul, (1, 1))


def kernel(ent_embeddings, rel_embeddings, pos_h, pos_r, pos_t,
           neg_h, neg_r, neg_t, take):
    bh = jnp.concatenate([pos_h, neg_h]).astype(jnp.int32)
    bt = jnp.concatenate([pos_t, neg_t]).astype(jnp.int32)
    br = jnp.concatenate([pos_r, neg_r]).astype(jnp.int32)
    take2 = jnp.concatenate([take, take]).astype(jnp.float32)

    eh, et, er = _gather3(ent_embeddings, rel_embeddings, bh, bt, br)

    loss2d, n_score = pl.pallas_call(
        _finish_body,
        out_shape=[
            jax.ShapeDtypeStruct((1, 1), jnp.float32),
            jax.ShapeDtypeStruct((BATCH,), jnp.float32),
        ],
    )(eh, et, er, take2)
    return loss2d[0, 0], n_score
